# baseline (speedup 1.0000x reference)
import functools

import jax
import jax.numpy as jnp
from jax import lax
from jax.experimental import pallas as pl
from jax.experimental.pallas import tpu as pltpu
from jax.experimental.pallas import tpu_sc as plsc

NC = 2
NS = 16
NW = NC * NS
CHUNK = 128
DEGW = 16

QF = 116
QS = 42
SLOW_CORE = 1
NCHT = NS * (QF + QS)

f32 = jnp.float32
i32 = jnp.int32


def _sc_mesh():
    return plsc.VectorSubcoreMesh(core_axis_name="c", subcore_axis_name="s")


_SC_PARAMS = pltpu.CompilerParams(use_tc_tiling_on_sc=False)


def _fill_rows(ref, nrows, width, value):
    def body(r, carry):
        for k in range(width // 16):
            ref[r, pl.ds(k * 16, 16)] = jnp.full((16,), value, f32)
        return carry
    lax.fori_loop(0, nrows, body, 0)


def _zero_stripe(zero_v, acc, sub, nacc):
    stripe = nacc // NS
    base = sub * stripe
    nfull = stripe // 128
    rem = stripe - nfull * 128
    for k in range(nfull):
        pltpu.sync_copy(zero_v, acc.at[pl.ds(base + k * 128, 128)])
    if rem:
        pltpu.sync_copy(zero_v.at[pl.ds(0, rem)], acc.at[pl.ds(base + nfull * 128, rem)])


def _stripe_rows(n):
    return (((n + NS - 1) // NS) + 7) // 8 * 8


def _chunk_base(c, s):
    if SLOW_CORE == 0:
        fast = c != 0
        return jnp.where(fast, NS * QS + s * QF, s * QS)
    fast = c != 1
    return jnp.where(fast, NS * QS + s * QF, s * QS)


def _tc_pad_edges(edge_index, n):
    e = edge_index.shape[1]
    erows = e // 128
    ei2 = edge_index.reshape(2, erows, 128)
    blk = 632
    grid = NCHT // blk

    def body(e_ref, s_ref, d_ref):
        b = pl.program_id(0)
        v = e_ref[...]
        row = jax.lax.broadcasted_iota(i32, (blk, 128), 0)
        col = jax.lax.broadcasted_iota(i32, (blk, 128), 1)
        flat = (b * blk + row) * 128 + col
        valid = flat < e
        s_ref[...] = jnp.where(valid, v[0], 0)
        d_ref[...] = jnp.where(valid, v[1], n)

    return pl.pallas_call(
        body,
        grid=(grid,),
        in_specs=[pl.BlockSpec((2, blk, 128), lambda b: (0, b, 0))],
        out_specs=[pl.BlockSpec((blk, 128), lambda b: (b, 0))] * 2,
        out_shape=[jax.ShapeDtypeStruct((NCHT, 128), i32)] * 2,
    )(ei2)


def _sc_degree(dstp, n):
    stripe = _stripe_rows(n)
    nacc = stripe * NS

    @functools.partial(
        pl.kernel,
        out_type=jax.ShapeDtypeStruct((NC, nacc, DEGW), f32),
        mesh=_sc_mesh(),
        scratch_types=[
            pltpu.VMEM((QF, CHUNK), i32),
            pltpu.VMEM((CHUNK, DEGW), f32),
            pltpu.VMEM((128, DEGW), f32),
            pltpu.VMEM_SHARED((nacc, DEGW), f32),
            pltpu.SemaphoreType.DMA,
        ],
        compiler_params=_SC_PARAMS,
    )
    def k(dst_hbm, out_hbm, idx_v, ones_v, zero_v, acc, sem):
        c = lax.axis_index("c")
        s = lax.axis_index("s")
        base = _chunk_base(c, s)
        _fill_rows(ones_v, CHUNK, DEGW, 1.0)
        _fill_rows(zero_v, 128, DEGW, 0.0)
        _zero_stripe(zero_v, acc, s, nacc)
        plsc.subcore_barrier()

        def scatter_chunks(cnt):
            pltpu.sync_copy(dst_hbm.at[pl.ds(base, cnt)], idx_v.at[pl.ds(0, cnt)])
            descs = []
            for j in range(cnt):
                descs.append(pltpu.async_copy(ones_v, acc.at[idx_v.at[j]], sem, add=True))
                if len(descs) == 8:
                    for dd in descs:
                        dd.wait()
                    descs = []
            for dd in descs:
                dd.wait()

        @pl.when(c == SLOW_CORE)
        def _():
            scatter_chunks(QS)

        @pl.when(c != SLOW_CORE)
        def _():
            scatter_chunks(QF)

        plsc.subcore_barrier()
        ob = s * stripe
        pltpu.sync_copy(acc.at[pl.ds(ob, stripe)], out_hbm.at[c, pl.ds(ob, stripe), :])

    return k(dstp)


def _sc_aggregate(gs, srcp, dstp):
    np_ = len(gs)
    n, d = gs[0].shape
    stripe = _stripe_rows(n)
    nacc = stripe * NS
    nbuf = 6
    ahead = 3

    @functools.partial(
        pl.kernel,
        out_type=jax.ShapeDtypeStruct((NC, np_, nacc, d), f32),
        mesh=_sc_mesh(),
        scratch_types=[
            pltpu.VMEM((QF, CHUNK), i32),
            pltpu.VMEM((QF, CHUNK), i32),
            pltpu.VMEM((nbuf, CHUNK, d), f32),
            pltpu.VMEM((128, d), f32),
            pltpu.VMEM_SHARED((nacc, d), f32),
            pltpu.SemaphoreType.DMA,
            pltpu.SemaphoreType.DMA,
        ],
        compiler_params=_SC_PARAMS,
    )
    def k(*refs):
        g_hbms = refs[:np_]
        src_hbm, dst_hbm, out_hbm = refs[np_:np_ + 3]
        src_v, dst_v, bufs, zero_v, acc, gsem, ssem = refs[np_ + 3:]
        c = lax.axis_index("c")
        s = lax.axis_index("s")
        base = _chunk_base(c, s)
        _fill_rows(zero_v, 128, d, 0.0)

        def load_idx(cnt):
            pltpu.sync_copy(src_hbm.at[pl.ds(base, cnt)], src_v.at[pl.ds(0, cnt)])
            pltpu.sync_copy(dst_hbm.at[pl.ds(base, cnt)], dst_v.at[pl.ds(0, cnt)])

        @pl.when(c == SLOW_CORE)
        def _():
            load_idx(QS)

        @pl.when(c != SLOW_CORE)
        def _():
            load_idx(QF)

        def pipeline(g_hbm, cnt):
            gd = {}
            sd = {}
            for j in range(min(ahead, cnt)):
                gd[j] = pltpu.async_copy(g_hbm.at[src_v.at[j]], bufs.at[j % nbuf], gsem)
            for j in range(cnt):
                m = j + ahead
                if m < cnt:
                    if m - nbuf >= 0:
                        sd[m - nbuf].wait()
                    gd[m] = pltpu.async_copy(
                        g_hbm.at[src_v.at[m]], bufs.at[m % nbuf], gsem)
                gd[j].wait()
                sd[j] = pltpu.async_copy(bufs.at[j % nbuf], acc.at[dst_v.at[j]],
                                         ssem, add=True)
            for j in range(max(0, cnt - nbuf), cnt):
                sd[j].wait()

        for p in range(np_):
            g_hbm = g_hbms[p]
            _zero_stripe(zero_v, acc, s, nacc)
            plsc.subcore_barrier()

            @pl.when(c == SLOW_CORE)
            def _():
                pipeline(g_hbm, QS)

            @pl.when(c != SLOW_CORE)
            def _():
                pipeline(g_hbm, QF)

            plsc.subcore_barrier()
            ob = s * stripe
            pltpu.sync_copy(acc.at[pl.ds(ob, stripe)],
                            out_hbm.at[c, p, pl.ds(ob, stripe), :])
            if p + 1 < np_:
                plsc.subcore_barrier()

    return k(*gs, srcp, dstp)


def _dinv_block(p_ref):
    deg = p_ref[0, :, 0:1] + p_ref[1, :, 0:1] + 1.0
    return lax.rsqrt(jnp.maximum(deg, 1.0))


def _tc_project1(P, x, W1):
    n, nf = x.shape
    nh = W1.shape[1]
    r = 1000
    grid = n // r

    def body(p_ref, x_ref, w_ref, z_ref, ga_ref, gb_ref):
        dinv = _dinv_block(p_ref)
        z = jnp.dot(x_ref[...], w_ref[...], preferred_element_type=f32,
                    precision=lax.Precision.HIGHEST)
        z_ref[...] = z
        g = z * dinv
        ga_ref[...] = g[:, : nh // 2]
        gb_ref[...] = g[:, nh // 2 :]

    return pl.pallas_call(
        body,
        grid=(grid,),
        in_specs=[
            pl.BlockSpec((2, r, DEGW), lambda i: (0, i, 0)),
            pl.BlockSpec((r, nf), lambda i: (i, 0)),
            pl.BlockSpec((nf, nh), lambda i: (0, 0)),
        ],
        out_specs=[pl.BlockSpec((r, nh), lambda i: (i, 0))]
        + [pl.BlockSpec((r, nh // 2), lambda i: (i, 0))] * 2,
        out_shape=[jax.ShapeDtypeStruct((n, nh), f32)]
        + [jax.ShapeDtypeStruct((n, nh // 2), f32)] * 2,
    )(P, x, W1)


def _tc_combine_project2(P, S, z1, b1r, W2p):
    n, nh = z1.shape
    d2 = W2p.shape[1]
    r = 1000
    grid = n // r

    def body(p_ref, s_ref, z1_ref, b_ref, w_ref, z2_ref, g2_ref):
        dinv = _dinv_block(p_ref)
        agg = jnp.concatenate(
            [s_ref[0, 0] + s_ref[1, 0], s_ref[0, 1] + s_ref[1, 1]], axis=1)
        out1 = dinv * agg + (dinv * dinv) * z1_ref[...] + b_ref[...]
        h2 = jnp.maximum(out1, 0.0)
        z2 = jnp.dot(h2, w_ref[...], preferred_element_type=f32,
                     precision=lax.Precision.HIGHEST)
        z2_ref[...] = z2
        g2_ref[...] = z2 * dinv

    return pl.pallas_call(
        body,
        grid=(grid,),
        in_specs=[
            pl.BlockSpec((2, r, DEGW), lambda i: (0, i, 0)),
            pl.BlockSpec((2, 2, r, nh // 2), lambda i: (0, 0, i, 0)),
            pl.BlockSpec((r, nh), lambda i: (i, 0)),
            pl.BlockSpec((1, nh), lambda i: (0, 0)),
            pl.BlockSpec((nh, d2), lambda i: (0, 0)),
        ],
        out_specs=[pl.BlockSpec((r, d2), lambda i: (i, 0))] * 2,
        out_shape=[jax.ShapeDtypeStruct((n, d2), f32)] * 2,
    )(P, S, z1, b1r, W2p)


def _tc_combine_out(P, Q, z2, b2r, nclass):
    n, d2 = z2.shape
    r = 1000
    grid = n // r

    def body(p_ref, q_ref, z2_ref, b_ref, o_ref):
        dinv = _dinv_block(p_ref)
        agg = q_ref[0, 0] + q_ref[1, 0]
        outf = dinv * agg + (dinv * dinv) * z2_ref[...] + b_ref[...]
        o_ref[...] = outf[:, :nclass]

    return pl.pallas_call(
        body,
        grid=(grid,),
        in_specs=[
            pl.BlockSpec((2, r, DEGW), lambda i: (0, i, 0)),
            pl.BlockSpec((2, 1, r, d2), lambda i: (0, 0, i, 0)),
            pl.BlockSpec((r, d2), lambda i: (i, 0)),
            pl.BlockSpec((1, d2), lambda i: (0, 0)),
        ],
        out_specs=pl.BlockSpec((r, nclass), lambda i: (i, 0)),
        out_shape=jax.ShapeDtypeStruct((n, nclass), f32),
    )(P, Q, z2, b2r)


def kernel(x, edge_index, W1, b1, W2, b2):
    n = x.shape[0]
    nclass = W2.shape[1]
    srcp, dstp = _tc_pad_edges(edge_index.astype(i32), n)

    d2 = -(-nclass // 16) * 16
    W2p = jnp.pad(W2, ((0, 0), (0, d2 - nclass)))
    b1r = b1.reshape(1, -1)
    b2r = jnp.pad(b2, (0, d2 - nclass)).reshape(1, -1)

    P = _sc_degree(dstp, n)
    z1, g1a, g1b = _tc_project1(P, x, W1)
    S = _sc_aggregate([g1a, g1b], srcp, dstp)
    z2, g2 = _tc_combine_project2(P, S, z1, b1r, W2p)
    Q = _sc_aggregate([g2], srcp, dstp)
    out = _tc_combine_out(P, Q, z2, b2r, nclass)
    return out


# speedup vs baseline: 1.0274x; 1.0274x over previous
import functools

import jax
import jax.numpy as jnp
from jax import lax
from jax.experimental import pallas as pl
from jax.experimental.pallas import tpu as pltpu
from jax.experimental.pallas import tpu_sc as plsc

NC = 2
NS = 16
NW = NC * NS
CHUNK = 128
DEGW = 16

QF = 116
QS = 42
SLOW_CORE = 0
NCHT = NS * (QF + QS)

f32 = jnp.float32
i32 = jnp.int32


def _sc_mesh():
    return plsc.VectorSubcoreMesh(core_axis_name="c", subcore_axis_name="s")


_SC_PARAMS = pltpu.CompilerParams(use_tc_tiling_on_sc=False)


def _fill_rows(ref, nrows, width, value):
    def body(r, carry):
        for k in range(width // 16):
            ref[r, pl.ds(k * 16, 16)] = jnp.full((16,), value, f32)
        return carry
    lax.fori_loop(0, nrows, body, 0)


def _zero_stripe(zero_v, acc, sub, nacc):
    stripe = nacc // NS
    base = sub * stripe
    nfull = stripe // 128
    rem = stripe - nfull * 128
    for k in range(nfull):
        pltpu.sync_copy(zero_v, acc.at[pl.ds(base + k * 128, 128)])
    if rem:
        pltpu.sync_copy(zero_v.at[pl.ds(0, rem)], acc.at[pl.ds(base + nfull * 128, rem)])


def _stripe_rows(n):
    return (((n + NS - 1) // NS) + 7) // 8 * 8


def _chunk_base(c, s):
    if SLOW_CORE == 0:
        fast = c != 0
        return jnp.where(fast, NS * QS + s * QF, s * QS)
    fast = c != 1
    return jnp.where(fast, NS * QS + s * QF, s * QS)


def _tc_pad_edges(edge_index, n):
    e = edge_index.shape[1]
    erows = e // 128
    ei2 = edge_index.reshape(2, erows, 128)
    blk = 632
    grid = NCHT // blk

    def body(e_ref, s_ref, d_ref):
        b = pl.program_id(0)
        v = e_ref[...]
        row = jax.lax.broadcasted_iota(i32, (blk, 128), 0)
        col = jax.lax.broadcasted_iota(i32, (blk, 128), 1)
        flat = (b * blk + row) * 128 + col
        valid = flat < e
        s_ref[...] = jnp.where(valid, v[0], 0)
        d_ref[...] = jnp.where(valid, v[1], n)

    return pl.pallas_call(
        body,
        grid=(grid,),
        in_specs=[pl.BlockSpec((2, blk, 128), lambda b: (0, b, 0))],
        out_specs=[pl.BlockSpec((blk, 128), lambda b: (b, 0))] * 2,
        out_shape=[jax.ShapeDtypeStruct((NCHT, 128), i32)] * 2,
    )(ei2)


def _sc_degree(dstp, n):
    stripe = _stripe_rows(n)
    nacc = stripe * NS

    @functools.partial(
        pl.kernel,
        out_type=jax.ShapeDtypeStruct((NC, nacc, DEGW), f32),
        mesh=_sc_mesh(),
        scratch_types=[
            pltpu.VMEM((QF, CHUNK), i32),
            pltpu.VMEM((CHUNK, DEGW), f32),
            pltpu.VMEM((128, DEGW), f32),
            pltpu.VMEM_SHARED((nacc, DEGW), f32),
            pltpu.SemaphoreType.DMA,
        ],
        compiler_params=_SC_PARAMS,
    )
    def k(dst_hbm, out_hbm, idx_v, ones_v, zero_v, acc, sem):
        c = lax.axis_index("c")
        s = lax.axis_index("s")
        base = _chunk_base(c, s)
        _fill_rows(ones_v, CHUNK, DEGW, 1.0)
        _fill_rows(zero_v, 128, DEGW, 0.0)
        _zero_stripe(zero_v, acc, s, nacc)
        plsc.subcore_barrier()

        def scatter_chunks(cnt):
            pltpu.sync_copy(dst_hbm.at[pl.ds(base, cnt)], idx_v.at[pl.ds(0, cnt)])
            descs = []
            for j in range(cnt):
                descs.append(pltpu.async_copy(ones_v, acc.at[idx_v.at[j]], sem, add=True))
                if len(descs) == 8:
                    for dd in descs:
                        dd.wait()
                    descs = []
            for dd in descs:
                dd.wait()

        @pl.when(c == SLOW_CORE)
        def _():
            scatter_chunks(QS)

        @pl.when(c != SLOW_CORE)
        def _():
            scatter_chunks(QF)

        plsc.subcore_barrier()
        ob = s * stripe
        pltpu.sync_copy(acc.at[pl.ds(ob, stripe)], out_hbm.at[c, pl.ds(ob, stripe), :])

    return k(dstp)


def _sc_aggregate(gs, srcp, dstp):
    np_ = len(gs)
    n, d = gs[0].shape
    stripe = _stripe_rows(n)
    nacc = stripe * NS
    nbuf = 6
    ahead = 3

    @functools.partial(
        pl.kernel,
        out_type=jax.ShapeDtypeStruct((NC, np_, nacc, d), f32),
        mesh=_sc_mesh(),
        scratch_types=[
            pltpu.VMEM((QF, CHUNK), i32),
            pltpu.VMEM((QF, CHUNK), i32),
            pltpu.VMEM((nbuf, CHUNK, d), f32),
            pltpu.VMEM((128, d), f32),
            pltpu.VMEM_SHARED((nacc, d), f32),
            pltpu.SemaphoreType.DMA,
            pltpu.SemaphoreType.DMA,
        ],
        compiler_params=_SC_PARAMS,
    )
    def k(*refs):
        g_hbms = refs[:np_]
        src_hbm, dst_hbm, out_hbm = refs[np_:np_ + 3]
        src_v, dst_v, bufs, zero_v, acc, gsem, ssem = refs[np_ + 3:]
        c = lax.axis_index("c")
        s = lax.axis_index("s")
        base = _chunk_base(c, s)
        _fill_rows(zero_v, 128, d, 0.0)

        def load_idx(cnt):
            pltpu.sync_copy(src_hbm.at[pl.ds(base, cnt)], src_v.at[pl.ds(0, cnt)])
            pltpu.sync_copy(dst_hbm.at[pl.ds(base, cnt)], dst_v.at[pl.ds(0, cnt)])

        @pl.when(c == SLOW_CORE)
        def _():
            load_idx(QS)

        @pl.when(c != SLOW_CORE)
        def _():
            load_idx(QF)

        def pipeline(g_hbm, cnt):
            gd = {}
            sd = {}
            for j in range(min(ahead, cnt)):
                gd[j] = pltpu.async_copy(g_hbm.at[src_v.at[j]], bufs.at[j % nbuf], gsem)
            for j in range(cnt):
                m = j + ahead
                if m < cnt:
                    if m - nbuf >= 0:
                        sd[m - nbuf].wait()
                    gd[m] = pltpu.async_copy(
                        g_hbm.at[src_v.at[m]], bufs.at[m % nbuf], gsem)
                gd[j].wait()
                sd[j] = pltpu.async_copy(bufs.at[j % nbuf], acc.at[dst_v.at[j]],
                                         ssem, add=True)
            for j in range(max(0, cnt - nbuf), cnt):
                sd[j].wait()

        for p in range(np_):
            g_hbm = g_hbms[p]
            _zero_stripe(zero_v, acc, s, nacc)
            plsc.subcore_barrier()

            @pl.when(c == SLOW_CORE)
            def _():
                pipeline(g_hbm, QS)

            @pl.when(c != SLOW_CORE)
            def _():
                pipeline(g_hbm, QF)

            plsc.subcore_barrier()
            ob = s * stripe
            pltpu.sync_copy(acc.at[pl.ds(ob, stripe)],
                            out_hbm.at[c, p, pl.ds(ob, stripe), :])
            if p + 1 < np_:
                plsc.subcore_barrier()

    return k(*gs, srcp, dstp)


def _dinv_block(p_ref):
    deg = p_ref[0, :, 0:1] + p_ref[1, :, 0:1] + 1.0
    return lax.rsqrt(jnp.maximum(deg, 1.0))


def _tc_project1(P, x, W1):
    n, nf = x.shape
    nh = W1.shape[1]
    r = 1000
    grid = n // r

    def body(p_ref, x_ref, w_ref, z_ref, ga_ref, gb_ref):
        dinv = _dinv_block(p_ref)
        z = jnp.dot(x_ref[...], w_ref[...], preferred_element_type=f32)
        z_ref[...] = z
        g = z * dinv
        ga_ref[...] = g[:, : nh // 2]
        gb_ref[...] = g[:, nh // 2 :]

    return pl.pallas_call(
        body,
        grid=(grid,),
        in_specs=[
            pl.BlockSpec((2, r, DEGW), lambda i: (0, i, 0)),
            pl.BlockSpec((r, nf), lambda i: (i, 0)),
            pl.BlockSpec((nf, nh), lambda i: (0, 0)),
        ],
        out_specs=[pl.BlockSpec((r, nh), lambda i: (i, 0))]
        + [pl.BlockSpec((r, nh // 2), lambda i: (i, 0))] * 2,
        out_shape=[jax.ShapeDtypeStruct((n, nh), f32)]
        + [jax.ShapeDtypeStruct((n, nh // 2), f32)] * 2,
    )(P, x, W1)


def _tc_combine_project2(P, S, z1, b1r, W2p):
    n, nh = z1.shape
    d2 = W2p.shape[1]
    r = 1000
    grid = n // r

    def body(p_ref, s_ref, z1_ref, b_ref, w_ref, z2_ref, g2_ref):
        dinv = _dinv_block(p_ref)
        agg = jnp.concatenate(
            [s_ref[0, 0] + s_ref[1, 0], s_ref[0, 1] + s_ref[1, 1]], axis=1)
        out1 = dinv * agg + (dinv * dinv) * z1_ref[...] + b_ref[...]
        h2 = jnp.maximum(out1, 0.0)
        z2 = jnp.dot(h2, w_ref[...], preferred_element_type=f32)
        z2_ref[...] = z2
        g2_ref[...] = z2 * dinv

    return pl.pallas_call(
        body,
        grid=(grid,),
        in_specs=[
            pl.BlockSpec((2, r, DEGW), lambda i: (0, i, 0)),
            pl.BlockSpec((2, 2, r, nh // 2), lambda i: (0, 0, i, 0)),
            pl.BlockSpec((r, nh), lambda i: (i, 0)),
            pl.BlockSpec((1, nh), lambda i: (0, 0)),
            pl.BlockSpec((nh, d2), lambda i: (0, 0)),
        ],
        out_specs=[pl.BlockSpec((r, d2), lambda i: (i, 0))] * 2,
        out_shape=[jax.ShapeDtypeStruct((n, d2), f32)] * 2,
    )(P, S, z1, b1r, W2p)


def _tc_combine_out(P, Q, z2, b2r, nclass):
    n, d2 = z2.shape
    r = 1000
    grid = n // r

    def body(p_ref, q_ref, z2_ref, b_ref, o_ref):
        dinv = _dinv_block(p_ref)
        agg = q_ref[0, 0] + q_ref[1, 0]
        outf = dinv * agg + (dinv * dinv) * z2_ref[...] + b_ref[...]
        o_ref[...] = outf[:, :nclass]

    return pl.pallas_call(
        body,
        grid=(grid,),
        in_specs=[
            pl.BlockSpec((2, r, DEGW), lambda i: (0, i, 0)),
            pl.BlockSpec((2, 1, r, d2), lambda i: (0, 0, i, 0)),
            pl.BlockSpec((r, d2), lambda i: (i, 0)),
            pl.BlockSpec((1, d2), lambda i: (0, 0)),
        ],
        out_specs=pl.BlockSpec((r, nclass), lambda i: (i, 0)),
        out_shape=jax.ShapeDtypeStruct((n, nclass), f32),
    )(P, Q, z2, b2r)


def kernel(x, edge_index, W1, b1, W2, b2):
    n = x.shape[0]
    nclass = W2.shape[1]
    srcp, dstp = _tc_pad_edges(edge_index.astype(i32), n)

    d2 = -(-nclass // 16) * 16
    W2p = jnp.pad(W2, ((0, 0), (0, d2 - nclass)))
    b1r = b1.reshape(1, -1)
    b2r = jnp.pad(b2, (0, d2 - nclass)).reshape(1, -1)

    P = _sc_degree(dstp, n)
    z1, g1a, g1b = _tc_project1(P, x, W1)
    S = _sc_aggregate([g1a, g1b], srcp, dstp)
    z2, g2 = _tc_combine_project2(P, S, z1, b1r, W2p)
    Q = _sc_aggregate([g2], srcp, dstp)
    out = _tc_combine_out(P, Q, z2, b2r, nclass)
    return out


# speedup vs baseline: 1.0867x; 1.0576x over previous
import functools

import jax
import jax.numpy as jnp
from jax import lax
from jax.experimental import pallas as pl
from jax.experimental.pallas import tpu as pltpu
from jax.experimental.pallas import tpu_sc as plsc

NC = 2
NS = 16
NW = NC * NS
CHUNK = 128
DEGW = 16

QF = 116
QS = 42
SLOW_CORE = 0
NCHT = NS * (QF + QS)

f32 = jnp.float32
i32 = jnp.int32


def _sc_mesh():
    return plsc.VectorSubcoreMesh(core_axis_name="c", subcore_axis_name="s")


_SC_PARAMS = pltpu.CompilerParams(use_tc_tiling_on_sc=False)


def _fill_rows(ref, nrows, width, value):
    def body(r, carry):
        for k in range(width // 16):
            ref[r, pl.ds(k * 16, 16)] = jnp.full((16,), value, f32)
        return carry
    lax.fori_loop(0, nrows, body, 0)


def _zero_stripe(zero_v, acc, sub, nacc):
    stripe = nacc // NS
    base = sub * stripe
    nfull = stripe // 128
    rem = stripe - nfull * 128
    for k in range(nfull):
        pltpu.sync_copy(zero_v, acc.at[pl.ds(base + k * 128, 128)])
    if rem:
        pltpu.sync_copy(zero_v.at[pl.ds(0, rem)], acc.at[pl.ds(base + nfull * 128, rem)])


def _stripe_rows(n):
    return (((n + NS - 1) // NS) + 7) // 8 * 8


def _chunk_base(c, s):
    if SLOW_CORE == 0:
        fast = c != 0
        return jnp.where(fast, NS * QS + s * QF, s * QS)
    fast = c != 1
    return jnp.where(fast, NS * QS + s * QF, s * QS)


def _tc_pad_edges(edge_index, n):
    e = edge_index.shape[1]
    erows = e // 128
    ei2 = edge_index.reshape(2, erows, 128)
    blk = 632
    grid = NCHT // blk

    def body(e_ref, s_ref, d_ref):
        b = pl.program_id(0)
        v = e_ref[...]
        row = jax.lax.broadcasted_iota(i32, (blk, 128), 0)
        col = jax.lax.broadcasted_iota(i32, (blk, 128), 1)
        flat = (b * blk + row) * 128 + col
        valid = flat < e
        s_ref[...] = jnp.where(valid, v[0], 0)
        d_ref[...] = jnp.where(valid, v[1], n)

    return pl.pallas_call(
        body,
        grid=(grid,),
        in_specs=[pl.BlockSpec((2, blk, 128), lambda b: (0, b, 0))],
        out_specs=[pl.BlockSpec((blk, 128), lambda b: (b, 0))] * 2,
        out_shape=[jax.ShapeDtypeStruct((NCHT, 128), i32)] * 2,
    )(ei2)


def _sc_degree(dstp, n):
    stripe = _stripe_rows(n)
    nacc = stripe * NS

    @functools.partial(
        pl.kernel,
        out_type=jax.ShapeDtypeStruct((NC, nacc, DEGW), f32),
        mesh=_sc_mesh(),
        scratch_types=[
            pltpu.VMEM((QF, CHUNK), i32),
            pltpu.VMEM((CHUNK, DEGW), f32),
            pltpu.VMEM((128, DEGW), f32),
            pltpu.VMEM_SHARED((nacc, DEGW), f32),
            pltpu.SemaphoreType.DMA,
        ],
        compiler_params=_SC_PARAMS,
    )
    def k(dst_hbm, out_hbm, idx_v, ones_v, zero_v, acc, sem):
        c = lax.axis_index("c")
        s = lax.axis_index("s")
        base = _chunk_base(c, s)
        _fill_rows(ones_v, CHUNK, DEGW, 1.0)
        _fill_rows(zero_v, 128, DEGW, 0.0)
        _zero_stripe(zero_v, acc, s, nacc)
        plsc.subcore_barrier()

        def scatter_chunks(cnt):
            pltpu.sync_copy(dst_hbm.at[pl.ds(base, cnt)], idx_v.at[pl.ds(0, cnt)])
            descs = []
            for j in range(cnt):
                descs.append(pltpu.async_copy(ones_v, acc.at[idx_v.at[j]], sem, add=True))
                if len(descs) == 8:
                    for dd in descs:
                        dd.wait()
                    descs = []
            for dd in descs:
                dd.wait()

        @pl.when(c == SLOW_CORE)
        def _():
            scatter_chunks(QS)

        @pl.when(c != SLOW_CORE)
        def _():
            scatter_chunks(QF)

        plsc.subcore_barrier()
        ob = s * stripe
        pltpu.sync_copy(acc.at[pl.ds(ob, stripe)], out_hbm.at[c, pl.ds(ob, stripe), :])

    return k(dstp)


def _sc_aggregate(gs, srcp, dstp):
    np_ = len(gs)
    n, d = gs[0].shape
    stripe = _stripe_rows(n)
    nacc = stripe * NS
    nbuf = 6
    ahead = 3

    @functools.partial(
        pl.kernel,
        out_type=jax.ShapeDtypeStruct((NC, nacc, 128), f32),
        mesh=_sc_mesh(),
        scratch_types=[
            pltpu.VMEM((QF, CHUNK), i32),
            pltpu.VMEM((QF, CHUNK), i32),
            pltpu.VMEM((nbuf, CHUNK, d), f32),
            pltpu.VMEM((128, d), f32),
            pltpu.VMEM_SHARED((nacc, d), f32),
            pltpu.SemaphoreType.DMA,
            pltpu.SemaphoreType.DMA,
        ],
        compiler_params=_SC_PARAMS,
    )
    def k(*refs):
        g_hbms = refs[:np_]
        src_hbm, dst_hbm, out_hbm = refs[np_:np_ + 3]
        src_v, dst_v, bufs, zero_v, acc, gsem, ssem = refs[np_ + 3:]
        c = lax.axis_index("c")
        s = lax.axis_index("s")
        base = _chunk_base(c, s)
        _fill_rows(zero_v, 128, d, 0.0)

        def load_idx(cnt):
            pltpu.sync_copy(src_hbm.at[pl.ds(base, cnt)], src_v.at[pl.ds(0, cnt)])
            pltpu.sync_copy(dst_hbm.at[pl.ds(base, cnt)], dst_v.at[pl.ds(0, cnt)])

        @pl.when(c == SLOW_CORE)
        def _():
            load_idx(QS)

        @pl.when(c != SLOW_CORE)
        def _():
            load_idx(QF)

        def pipeline(g_hbm, cnt):
            gd = {}
            sd = {}
            for j in range(min(ahead, cnt)):
                gd[j] = pltpu.async_copy(g_hbm.at[src_v.at[j]], bufs.at[j % nbuf], gsem)
            for j in range(cnt):
                m = j + ahead
                if m < cnt:
                    if m - nbuf >= 0:
                        sd[m - nbuf].wait()
                    gd[m] = pltpu.async_copy(
                        g_hbm.at[src_v.at[m]], bufs.at[m % nbuf], gsem)
                gd[j].wait()
                sd[j] = pltpu.async_copy(bufs.at[j % nbuf], acc.at[dst_v.at[j]],
                                         ssem, add=True)
            for j in range(max(0, cnt - nbuf), cnt):
                sd[j].wait()

        for p in range(np_):
            g_hbm = g_hbms[p]
            _zero_stripe(zero_v, acc, s, nacc)
            plsc.subcore_barrier()

            @pl.when(c == SLOW_CORE)
            def _():
                pipeline(g_hbm, QS)

            @pl.when(c != SLOW_CORE)
            def _():
                pipeline(g_hbm, QF)

            plsc.subcore_barrier()
            ob = s * stripe
            pltpu.sync_copy(acc.at[pl.ds(ob, stripe)],
                            out_hbm.at[c, pl.ds(ob, stripe), pl.ds(p * d, d)])
            if p + 1 < np_:
                plsc.subcore_barrier()

    return k(*gs, srcp, dstp)


def _dinv_block(p_ref):
    deg = p_ref[0, :, 0:1] + p_ref[1, :, 0:1] + 1.0
    return lax.rsqrt(jnp.maximum(deg, 1.0))


def _tc_project1(P, x, W1):
    n, nf = x.shape
    nh = W1.shape[1]
    r = 1000
    grid = n // r

    def body(p_ref, x_ref, w_ref, z_ref, ga_ref, gb_ref):
        dinv = _dinv_block(p_ref)
        z = jnp.dot(x_ref[...], w_ref[...], preferred_element_type=f32)
        z_ref[...] = z
        g = z * dinv
        ga_ref[...] = g[:, : nh // 2]
        gb_ref[...] = g[:, nh // 2 :]

    return pl.pallas_call(
        body,
        grid=(grid,),
        in_specs=[
            pl.BlockSpec((2, r, DEGW), lambda i: (0, i, 0)),
            pl.BlockSpec((r, nf), lambda i: (i, 0)),
            pl.BlockSpec((nf, nh), lambda i: (0, 0)),
        ],
        out_specs=[pl.BlockSpec((r, nh), lambda i: (i, 0))]
        + [pl.BlockSpec((r, nh // 2), lambda i: (i, 0))] * 2,
        out_shape=[jax.ShapeDtypeStruct((n, nh), f32)]
        + [jax.ShapeDtypeStruct((n, nh // 2), f32)] * 2,
    )(P, x, W1)


def _tc_combine_project2(P, S, z1, b1r, W2p):
    n, nh = z1.shape
    d2 = W2p.shape[1]
    r = 1000
    grid = n // r

    def body(p_ref, s_ref, z1_ref, b_ref, w_ref, z2_ref, g2_ref):
        dinv = _dinv_block(p_ref)
        agg = s_ref[0] + s_ref[1]
        out1 = dinv * agg + (dinv * dinv) * z1_ref[...] + b_ref[...]
        h2 = jnp.maximum(out1, 0.0)
        z2 = jnp.dot(h2, w_ref[...], preferred_element_type=f32)
        z2_ref[...] = z2
        g2_ref[...] = z2 * dinv

    return pl.pallas_call(
        body,
        grid=(grid,),
        in_specs=[
            pl.BlockSpec((2, r, DEGW), lambda i: (0, i, 0)),
            pl.BlockSpec((2, r, nh), lambda i: (0, i, 0)),
            pl.BlockSpec((r, nh), lambda i: (i, 0)),
            pl.BlockSpec((1, nh), lambda i: (0, 0)),
            pl.BlockSpec((nh, d2), lambda i: (0, 0)),
        ],
        out_specs=[pl.BlockSpec((r, d2), lambda i: (i, 0))] * 2,
        out_shape=[jax.ShapeDtypeStruct((n, d2), f32)] * 2,
    )(P, S, z1, b1r, W2p)


def _tc_combine_out(P, Q, z2, b2r, nclass):
    n, d2 = z2.shape
    r = 1000
    grid = n // r

    def body(p_ref, q_ref, z2_ref, b_ref, o_ref):
        dinv = _dinv_block(p_ref)
        agg = q_ref[0, :, :d2] + q_ref[1, :, :d2]
        outf = dinv * agg + (dinv * dinv) * z2_ref[...] + b_ref[...]
        o_ref[...] = outf[:, :nclass]

    return pl.pallas_call(
        body,
        grid=(grid,),
        in_specs=[
            pl.BlockSpec((2, r, DEGW), lambda i: (0, i, 0)),
            pl.BlockSpec((2, r, 128), lambda i: (0, i, 0)),
            pl.BlockSpec((r, d2), lambda i: (i, 0)),
            pl.BlockSpec((1, d2), lambda i: (0, 0)),
        ],
        out_specs=pl.BlockSpec((r, nclass), lambda i: (i, 0)),
        out_shape=jax.ShapeDtypeStruct((n, nclass), f32),
    )(P, Q, z2, b2r)


def kernel(x, edge_index, W1, b1, W2, b2):
    n = x.shape[0]
    nclass = W2.shape[1]
    srcp, dstp = _tc_pad_edges(edge_index.astype(i32), n)

    d2 = -(-nclass // 16) * 16
    W2p = jnp.pad(W2, ((0, 0), (0, d2 - nclass)))
    b1r = b1.reshape(1, -1)
    b2r = jnp.pad(b2, (0, d2 - nclass)).reshape(1, -1)

    P = _sc_degree(dstp, n)
    z1, g1a, g1b = _tc_project1(P, x, W1)
    S = _sc_aggregate([g1a, g1b], srcp, dstp)
    z2, g2 = _tc_combine_project2(P, S, z1, b1r, W2p)
    Q = _sc_aggregate([g2], srcp, dstp)
    out = _tc_combine_out(P, Q, z2, b2r, nclass)
    return out


# speedup vs baseline: 1.0869x; 1.0002x over previous
import functools

import jax
import jax.numpy as jnp
from jax import lax
from jax.experimental import pallas as pl
from jax.experimental.pallas import tpu as pltpu
from jax.experimental.pallas import tpu_sc as plsc

NC = 2
NS = 16
NW = NC * NS
CHUNK = 128
DEGW = 16

QF = 116
QS = 42
SLOW_CORE = 0
NCHT = NS * (QF + QS)

f32 = jnp.float32
i32 = jnp.int32


def _sc_mesh():
    return plsc.VectorSubcoreMesh(core_axis_name="c", subcore_axis_name="s")


_SC_PARAMS = pltpu.CompilerParams(use_tc_tiling_on_sc=False)


def _fill_rows(ref, nrows, width, value):
    def body(r, carry):
        for k in range(width // 16):
            ref[r, pl.ds(k * 16, 16)] = jnp.full((16,), value, f32)
        return carry
    lax.fori_loop(0, nrows, body, 0)


def _zero_stripe(zero_v, acc, sub, nacc):
    stripe = nacc // NS
    base = sub * stripe
    nfull = stripe // 128
    rem = stripe - nfull * 128
    for k in range(nfull):
        pltpu.sync_copy(zero_v, acc.at[pl.ds(base + k * 128, 128)])
    if rem:
        pltpu.sync_copy(zero_v.at[pl.ds(0, rem)], acc.at[pl.ds(base + nfull * 128, rem)])


def _stripe_rows(n):
    return (((n + NS - 1) // NS) + 7) // 8 * 8


def _chunk_base(c, s):
    if SLOW_CORE == 0:
        fast = c != 0
        return jnp.where(fast, NS * QS + s * QF, s * QS)
    fast = c != 1
    return jnp.where(fast, NS * QS + s * QF, s * QS)


def _tc_pad_edges(edge_index, n):
    e = edge_index.shape[1]
    erows = e // 128
    ei2 = edge_index.reshape(2, erows, 128)
    blk = 632
    grid = NCHT // blk

    def body(e_ref, s_ref, d_ref):
        b = pl.program_id(0)
        v = e_ref[...]
        row = jax.lax.broadcasted_iota(i32, (blk, 128), 0)
        col = jax.lax.broadcasted_iota(i32, (blk, 128), 1)
        flat = (b * blk + row) * 128 + col
        valid = flat < e
        s_ref[...] = jnp.where(valid, v[0], 0)
        d_ref[...] = jnp.where(valid, v[1], n)

    return pl.pallas_call(
        body,
        grid=(grid,),
        in_specs=[pl.BlockSpec((2, blk, 128), lambda b: (0, b, 0))],
        out_specs=[pl.BlockSpec((blk, 128), lambda b: (b, 0))] * 2,
        out_shape=[jax.ShapeDtypeStruct((NCHT, 128), i32)] * 2,
    )(ei2)


def _sc_degree(dstp, n):
    stripe = _stripe_rows(n)
    nacc = stripe * NS

    @functools.partial(
        pl.kernel,
        out_type=jax.ShapeDtypeStruct((NC, nacc, DEGW), f32),
        mesh=_sc_mesh(),
        scratch_types=[
            pltpu.VMEM((QF, CHUNK), i32),
            pltpu.VMEM((CHUNK, DEGW), f32),
            pltpu.VMEM((128, DEGW), f32),
            pltpu.VMEM_SHARED((nacc, DEGW), f32),
            pltpu.SemaphoreType.DMA,
        ],
        compiler_params=_SC_PARAMS,
    )
    def k(dst_hbm, out_hbm, idx_v, ones_v, zero_v, acc, sem):
        c = lax.axis_index("c")
        s = lax.axis_index("s")
        base = _chunk_base(c, s)
        _fill_rows(ones_v, CHUNK, DEGW, 1.0)
        _fill_rows(zero_v, 128, DEGW, 0.0)
        _zero_stripe(zero_v, acc, s, nacc)
        plsc.subcore_barrier()

        def scatter_chunks(cnt):
            pltpu.sync_copy(dst_hbm.at[pl.ds(base, cnt)], idx_v.at[pl.ds(0, cnt)])
            descs = []
            for j in range(cnt):
                descs.append(pltpu.async_copy(ones_v, acc.at[idx_v.at[j]], sem, add=True))
                if len(descs) == 16:
                    for dd in descs:
                        dd.wait()
                    descs = []
            for dd in descs:
                dd.wait()

        @pl.when(c == SLOW_CORE)
        def _():
            scatter_chunks(QS)

        @pl.when(c != SLOW_CORE)
        def _():
            scatter_chunks(QF)

        plsc.subcore_barrier()
        ob = s * stripe
        pltpu.sync_copy(acc.at[pl.ds(ob, stripe)], out_hbm.at[c, pl.ds(ob, stripe), :])

    return k(dstp)


def _sc_aggregate(gs, srcp, dstp):
    np_ = len(gs)
    n, d = gs[0].shape
    stripe = _stripe_rows(n)
    nacc = stripe * NS
    nbuf = 6
    ahead = 4

    @functools.partial(
        pl.kernel,
        out_type=jax.ShapeDtypeStruct((NC, nacc, 128), f32),
        mesh=_sc_mesh(),
        scratch_types=[
            pltpu.VMEM((QF, CHUNK), i32),
            pltpu.VMEM((QF, CHUNK), i32),
            pltpu.VMEM((nbuf, CHUNK, d), f32),
            pltpu.VMEM((128, d), f32),
            pltpu.VMEM_SHARED((nacc, d), f32),
            pltpu.SemaphoreType.DMA,
            pltpu.SemaphoreType.DMA,
        ],
        compiler_params=_SC_PARAMS,
    )
    def k(*refs):
        g_hbms = refs[:np_]
        src_hbm, dst_hbm, out_hbm = refs[np_:np_ + 3]
        src_v, dst_v, bufs, zero_v, acc, gsem, ssem = refs[np_ + 3:]
        c = lax.axis_index("c")
        s = lax.axis_index("s")
        base = _chunk_base(c, s)
        _fill_rows(zero_v, 128, d, 0.0)

        def load_idx(cnt):
            pltpu.sync_copy(src_hbm.at[pl.ds(base, cnt)], src_v.at[pl.ds(0, cnt)])
            pltpu.sync_copy(dst_hbm.at[pl.ds(base, cnt)], dst_v.at[pl.ds(0, cnt)])

        @pl.when(c == SLOW_CORE)
        def _():
            load_idx(QS)

        @pl.when(c != SLOW_CORE)
        def _():
            load_idx(QF)

        def pipeline(g_hbm, cnt):
            gd = {}
            sd = {}
            for j in range(min(ahead, cnt)):
                gd[j] = pltpu.async_copy(g_hbm.at[src_v.at[j]], bufs.at[j % nbuf], gsem)
            for j in range(cnt):
                m = j + ahead
                if m < cnt:
                    if m - nbuf >= 0:
                        sd[m - nbuf].wait()
                    gd[m] = pltpu.async_copy(
                        g_hbm.at[src_v.at[m]], bufs.at[m % nbuf], gsem)
                gd[j].wait()
                sd[j] = pltpu.async_copy(bufs.at[j % nbuf], acc.at[dst_v.at[j]],
                                         ssem, add=True)
            for j in range(max(0, cnt - nbuf), cnt):
                sd[j].wait()

        for p in range(np_):
            g_hbm = g_hbms[p]
            _zero_stripe(zero_v, acc, s, nacc)
            plsc.subcore_barrier()

            @pl.when(c == SLOW_CORE)
            def _():
                pipeline(g_hbm, QS)

            @pl.when(c != SLOW_CORE)
            def _():
                pipeline(g_hbm, QF)

            plsc.subcore_barrier()
            ob = s * stripe
            pltpu.sync_copy(acc.at[pl.ds(ob, stripe)],
                            out_hbm.at[c, pl.ds(ob, stripe), pl.ds(p * d, d)])
            if p + 1 < np_:
                plsc.subcore_barrier()

    return k(*gs, srcp, dstp)


def _dinv_block(p_ref):
    deg = p_ref[0, :, 0:1] + p_ref[1, :, 0:1] + 1.0
    return lax.rsqrt(jnp.maximum(deg, 1.0))


def _tc_project1(P, x, W1):
    n, nf = x.shape
    nh = W1.shape[1]
    r = 1000
    grid = n // r

    def body(p_ref, x_ref, w_ref, z_ref, ga_ref, gb_ref):
        dinv = _dinv_block(p_ref)
        z = jnp.dot(x_ref[...], w_ref[...], preferred_element_type=f32)
        z_ref[...] = z
        g = z * dinv
        ga_ref[...] = g[:, : nh // 2]
        gb_ref[...] = g[:, nh // 2 :]

    return pl.pallas_call(
        body,
        grid=(grid,),
        in_specs=[
            pl.BlockSpec((2, r, DEGW), lambda i: (0, i, 0)),
            pl.BlockSpec((r, nf), lambda i: (i, 0)),
            pl.BlockSpec((nf, nh), lambda i: (0, 0)),
        ],
        out_specs=[pl.BlockSpec((r, nh), lambda i: (i, 0))]
        + [pl.BlockSpec((r, nh // 2), lambda i: (i, 0))] * 2,
        out_shape=[jax.ShapeDtypeStruct((n, nh), f32)]
        + [jax.ShapeDtypeStruct((n, nh // 2), f32)] * 2,
    )(P, x, W1)


def _tc_combine_project2(P, S, z1, b1r, W2p):
    n, nh = z1.shape
    d2 = W2p.shape[1]
    r = 1000
    grid = n // r

    def body(p_ref, s_ref, z1_ref, b_ref, w_ref, z2_ref, g2_ref):
        dinv = _dinv_block(p_ref)
        agg = s_ref[0] + s_ref[1]
        out1 = dinv * agg + (dinv * dinv) * z1_ref[...] + b_ref[...]
        h2 = jnp.maximum(out1, 0.0)
        z2 = jnp.dot(h2, w_ref[...], preferred_element_type=f32)
        z2_ref[...] = z2
        g2_ref[...] = z2 * dinv

    return pl.pallas_call(
        body,
        grid=(grid,),
        in_specs=[
            pl.BlockSpec((2, r, DEGW), lambda i: (0, i, 0)),
            pl.BlockSpec((2, r, nh), lambda i: (0, i, 0)),
            pl.BlockSpec((r, nh), lambda i: (i, 0)),
            pl.BlockSpec((1, nh), lambda i: (0, 0)),
            pl.BlockSpec((nh, d2), lambda i: (0, 0)),
        ],
        out_specs=[pl.BlockSpec((r, d2), lambda i: (i, 0))] * 2,
        out_shape=[jax.ShapeDtypeStruct((n, d2), f32)] * 2,
    )(P, S, z1, b1r, W2p)


def _tc_combine_out(P, Q, z2, b2r, nclass):
    n, d2 = z2.shape
    r = 1000
    grid = n // r

    def body(p_ref, q_ref, z2_ref, b_ref, o_ref):
        dinv = _dinv_block(p_ref)
        agg = q_ref[0, :, :d2] + q_ref[1, :, :d2]
        outf = dinv * agg + (dinv * dinv) * z2_ref[...] + b_ref[...]
        o_ref[...] = outf[:, :nclass]

    return pl.pallas_call(
        body,
        grid=(grid,),
        in_specs=[
            pl.BlockSpec((2, r, DEGW), lambda i: (0, i, 0)),
            pl.BlockSpec((2, r, 128), lambda i: (0, i, 0)),
            pl.BlockSpec((r, d2), lambda i: (i, 0)),
            pl.BlockSpec((1, d2), lambda i: (0, 0)),
        ],
        out_specs=pl.BlockSpec((r, nclass), lambda i: (i, 0)),
        out_shape=jax.ShapeDtypeStruct((n, nclass), f32),
    )(P, Q, z2, b2r)


def kernel(x, edge_index, W1, b1, W2, b2):
    n = x.shape[0]
    nclass = W2.shape[1]
    srcp, dstp = _tc_pad_edges(edge_index.astype(i32), n)

    d2 = -(-nclass // 16) * 16
    W2p = jnp.pad(W2, ((0, 0), (0, d2 - nclass)))
    b1r = b1.reshape(1, -1)
    b2r = jnp.pad(b2, (0, d2 - nclass)).reshape(1, -1)

    P = _sc_degree(dstp, n)
    z1, g1a, g1b = _tc_project1(P, x, W1)
    S = _sc_aggregate([g1a, g1b], srcp, dstp)
    z2, g2 = _tc_combine_project2(P, S, z1, b1r, W2p)
    Q = _sc_aggregate([g2], srcp, dstp)
    out = _tc_combine_out(P, Q, z2, b2r, nclass)
    return out
